# baseline (speedup 1.0000x reference)
import functools

import numpy as np

import jax
import jax.numpy as jnp
from jax import lax
from jax.experimental import pallas as pl
from jax.experimental.pallas import tpu as pltpu
from jax.experimental.pallas import tpu_sc as plsc

_GROUPS = 4
_LANES = 16
_NC, _NS = 2, 16
_NW = _NC * _NS


def _sc_maxrel(xT, idx_j, idx_i, M, C, K):
    E = M * K
    EC = 128
    Q = 4
    NPC = EC // K
    NCH = E // EC
    TQ = -(-NCH // (_NW * Q))
    NCHP = _NW * TQ * Q
    Mp = NCHP * NPC
    CW = C // 2

    ij2 = jnp.zeros((NCHP, EC), jnp.int32).at[:NCH].set(idx_j.reshape(NCH, EC))
    ii2 = jnp.zeros((NCHP, EC), jnp.int32).at[:NCH].set(idx_i.reshape(NCH, EC))
    icat = jnp.stack([ij2, ii2], axis=1).reshape(NCHP // Q, Q, 2, EC)

    mesh = plsc.VectorSubcoreMesh(core_axis_name="c", subcore_axis_name="s")

    @functools.partial(
        pl.kernel,
        mesh=mesh,
        compiler_params=pltpu.CompilerParams(use_tc_tiling_on_sc=False),
        out_type=jax.ShapeDtypeStruct((Mp, C), jnp.float32),
        scratch_types=[
            pltpu.VMEM((Q, 2, EC), jnp.int32),
            pltpu.VMEM((Q, EC, CW), jnp.int32),
            pltpu.VMEM((Q, EC, CW), jnp.int32),
            pltpu.VMEM((Q * NPC, C), jnp.float32),
            pltpu.SemaphoreType.DMA,
            pltpu.SemaphoreType.DMA,
        ],
    )
    def sc_kernel(xT_hbm, ic_hbm, out_hbm, idx_v, rj_v, ri_v, o_v, semj, semi):
        wid = lax.axis_index("s") * _NC + lax.axis_index("c")

        MASK = jnp.int32(-65536)

        def halves(v):
            lo = lax.bitcast_convert_type(v << 16, jnp.float32)
            hi = lax.bitcast_convert_type(v & MASK, jnp.float32)
            return lo, hi

        def compute(q):
            def node(n, c2):
                for cc in range(CW // _LANES):
                    sl = pl.ds(cc * _LANES, _LANES)
                    je, jo = halves(rj_v[q, n * K, sl])
                    ie, io = halves(ri_v[q, n * K, sl])
                    acc_e = je - ie
                    acc_o = jo - io
                    for kk in range(1, K):
                        je, jo = halves(rj_v[q, n * K + kk, sl])
                        ie, io = halves(ri_v[q, n * K + kk, sl])
                        acc_e = jnp.maximum(acc_e, je - ie)
                        acc_o = jnp.maximum(acc_o, jo - io)
                    o_v[q * NPC + n, pl.ds(cc * 2 * _LANES, _LANES)] = acc_e
                    o_v[q * NPC + n, pl.ds(cc * 2 * _LANES + _LANES, _LANES)] = acc_o
                return c2
            lax.fori_loop(0, NPC, node, 0)

        def body(t, carry):
            it = wid + t * _NW
            pltpu.sync_copy(ic_hbm.at[it], idx_v)
            copies = []
            for q in range(Q):
                copies.append(pltpu.async_copy(
                    xT_hbm.at[idx_v.at[q, 0]], rj_v.at[q], semj))
                copies.append(pltpu.async_copy(
                    xT_hbm.at[idx_v.at[q, 1]], ri_v.at[q], semi))
            for cp in copies:
                cp.wait()
            for q in range(Q):
                compute(q)
            pltpu.sync_copy(o_v, out_hbm.at[pl.ds(it * Q * NPC, Q * NPC)])
            return carry

        lax.fori_loop(0, TQ, body, 0)

    out = sc_kernel(xT, icat)
    return out[:M]


def _tc_body(wx_ref, wj_ref, b_ref, x_ref, mr_ref, o_ref):
    xb = x_ref[0]
    mr = mr_ref[0]
    acc = jnp.dot(wx_ref[...], xb, preferred_element_type=jnp.float32)
    acc = acc + lax.dot_general(
        wj_ref[...], mr, (((1,), (1,)), ((), ())),
        preferred_element_type=jnp.float32)
    o_ref[0] = jnp.maximum(acc + b_ref[...], 0.0)


def _tc_conv(x3, mr3, Wx, Wj, bias):
    B, C, N = x3.shape
    COUT = Wx.shape[0]
    grid = (B,)
    return pl.pallas_call(
        _tc_body,
        grid=grid,
        in_specs=[
            pl.BlockSpec((COUT, C), lambda b: (0, 0)),
            pl.BlockSpec((COUT, C), lambda b: (0, 0)),
            pl.BlockSpec((COUT, 1), lambda b: (0, 0)),
            pl.BlockSpec((1, C, N), lambda b: (b, 0, 0)),
            pl.BlockSpec((1, N, C), lambda b: (b, 0, 0)),
        ],
        out_specs=pl.BlockSpec((1, COUT, N), lambda b: (b, 0, 0)),
        out_shape=jax.ShapeDtypeStruct((B, COUT, N), jnp.float32),
    )(Wx, Wj, bias.reshape(COUT, 1), x3, mr3)


def _block_diag(blocks):
    G, R, S = blocks.shape
    out = jnp.zeros((G * R, G * S), blocks.dtype)
    for g in range(G):
        out = out.at[g * R:(g + 1) * R, g * S:(g + 1) * S].set(blocks[g])
    return out


def kernel(x, edge_index, W, bias):
    B, C, N, _ = x.shape
    K = edge_index.shape[-1]
    COUT = W.shape[0]

    x3 = x[..., 0]
    xT = jnp.transpose(x3, (0, 2, 1)).reshape(B * N, C)
    ei = edge_index.astype(jnp.int32)
    base = (jnp.arange(B, dtype=jnp.int32) * N)[:, None, None]
    idx_j = (ei[0] + base).reshape(B * N * K)
    idx_i = (ei[1] + base).reshape(B * N * K)

    xT32 = lax.bitcast_convert_type(
        xT.astype(jnp.bfloat16).reshape(B * N, C // 2, 2), jnp.int32)
    mr = _sc_maxrel(xT32, idx_j, idx_i, B * N, C, K)

    Wg = W[:, :, 0, 0].reshape(_GROUPS, COUT // _GROUPS, (2 * C) // _GROUPS)
    Wx = _block_diag(Wg[:, :, 0::2])
    Wj = _block_diag(Wg[:, :, 1::2])
    blk = np.arange(C).reshape(C // 32, 16, 2)
    order = np.concatenate([blk[:, :, 0], blk[:, :, 1]], axis=1).reshape(-1)
    Wj = Wj[:, order]

    out = _tc_conv(x3, mr.reshape(B, N, C), Wx, Wj, bias)
    return out[..., None]


# speedup vs baseline: 1.9482x; 1.9482x over previous
import functools

import numpy as np

import jax
import jax.numpy as jnp
from jax import lax
from jax.experimental import pallas as pl
from jax.experimental.pallas import tpu as pltpu
from jax.experimental.pallas import tpu_sc as plsc

_GROUPS = 4
_LANES = 16
_NC, _NS = 2, 16
_WPT = 4


def _sc_maxrel(xTt, ijT, iiT, B, N, C, K, NP):
    S = 1024
    NCH = NP // S

    mesh = plsc.VectorSubcoreMesh(core_axis_name="c", subcore_axis_name="s")

    @functools.partial(
        pl.kernel,
        mesh=mesh,
        compiler_params=pltpu.CompilerParams(
            use_tc_tiling_on_sc=False, needs_layout_passes=False),
        out_type=jax.ShapeDtypeStruct((B, _NS, 2 * _WPT, NP), jnp.float32),
        scratch_types=[
            pltpu.VMEM((N * _WPT,), jnp.int32),
            pltpu.VMEM((K, S), jnp.int32),
            pltpu.VMEM((K, S), jnp.int32),
            pltpu.VMEM((2 * _WPT, S), jnp.float32),
            pltpu.SemaphoreType.DMA,
        ],
    )
    def sc_kernel(xTt_hbm, ij_hbm, ii_hbm, out_hbm, tbl_v, ij_v, ii_v, o_v, sem):
        b = lax.axis_index("c")
        h = lax.axis_index("s")

        pltpu.sync_copy(xTt_hbm.at[b, h], tbl_v)

        MASK = jnp.int32(-65536)

        def halves(v):
            lo = lax.bitcast_convert_type(v << 16, jnp.float32)
            hi = lax.bitcast_convert_type(v & MASK, jnp.float32)
            return lo, hi

        def chunk(t, carry):
            n0 = pl.multiple_of(t * S, S)
            pltpu.sync_copy(ij_hbm.at[b, :, pl.ds(n0, S)], ij_v)
            pltpu.sync_copy(ii_hbm.at[b, :, pl.ds(n0, S)], ii_v)

            def grp(g, c2):
                sl = pl.ds(pl.multiple_of(g * _LANES, _LANES), _LANES)
                acc_e = [None] * _WPT
                acc_o = [None] * _WPT
                for k in range(K):
                    bj = ij_v[k, sl] * _WPT
                    bi = ii_v[k, sl] * _WPT
                    for w in range(_WPT):
                        vj = plsc.load_gather(tbl_v, [bj + w])
                        vi = plsc.load_gather(tbl_v, [bi + w])
                        je, jo = halves(vj)
                        ie, io = halves(vi)
                        de = je - ie
                        do = jo - io
                        if k == 0:
                            acc_e[w] = de
                            acc_o[w] = do
                        else:
                            acc_e[w] = jnp.maximum(acc_e[w], de)
                            acc_o[w] = jnp.maximum(acc_o[w], do)
                for w in range(_WPT):
                    o_v[w, sl] = acc_e[w]
                    o_v[_WPT + w, sl] = acc_o[w]
                return c2

            lax.fori_loop(0, S // _LANES, grp, 0)
            pltpu.sync_copy(o_v, out_hbm.at[b, h, :, pl.ds(n0, S)])
            return carry

        lax.fori_loop(0, NCH, chunk, 0)

    return sc_kernel(xTt, ijT, iiT)


def _tc_body(wx_ref, wj_ref, b_ref, x_ref, mr_ref, o_ref):
    xb = x_ref[0]
    mr = mr_ref[0]
    acc = jnp.dot(wx_ref[...], xb, preferred_element_type=jnp.float32)
    acc = acc + jnp.dot(wj_ref[...], mr, preferred_element_type=jnp.float32)
    o_ref[0] = jnp.maximum(acc + b_ref[...], 0.0)


def _tc_conv(x3, mr3, Wx, Wj, bias):
    B, C, N = x3.shape
    COUT = Wx.shape[0]
    return pl.pallas_call(
        _tc_body,
        grid=(B,),
        in_specs=[
            pl.BlockSpec((COUT, C), lambda b: (0, 0)),
            pl.BlockSpec((COUT, C), lambda b: (0, 0)),
            pl.BlockSpec((COUT, 1), lambda b: (0, 0)),
            pl.BlockSpec((1, C, N), lambda b: (b, 0, 0)),
            pl.BlockSpec((1, C, N), lambda b: (b, 0, 0)),
        ],
        out_specs=pl.BlockSpec((1, COUT, N), lambda b: (b, 0, 0)),
        out_shape=jax.ShapeDtypeStruct((B, COUT, N), jnp.float32),
    )(Wx, Wj, bias.reshape(COUT, 1), x3, mr3)


def _block_diag(blocks):
    G, R, S = blocks.shape
    out = jnp.zeros((G * R, G * S), blocks.dtype)
    for g in range(G):
        out = out.at[g * R:(g + 1) * R, g * S:(g + 1) * S].set(blocks[g])
    return out


def kernel(x, edge_index, W, bias):
    B, C, N, _ = x.shape
    K = edge_index.shape[-1]
    COUT = W.shape[0]
    S = 1024
    NP = -(-N // S) * S

    x3 = x[..., 0]
    xT = jnp.transpose(x3, (0, 2, 1))
    xT32 = lax.bitcast_convert_type(
        xT.astype(jnp.bfloat16).reshape(B, N, C // 2, 2), jnp.int32)
    xTt = (xT32.reshape(B, N, _NS, _WPT)
           .transpose(0, 2, 1, 3).reshape(B, _NS, N * _WPT))

    ei = edge_index.astype(jnp.int32)
    pad = [(0, 0), (0, NP - N), (0, 0)]
    ijT = jnp.transpose(jnp.pad(ei[0], pad), (0, 2, 1))
    iiT = jnp.transpose(jnp.pad(ei[1], pad), (0, 2, 1))

    mr = _sc_maxrel(xTt, ijT, iiT, B, N, C, K, NP)
    mr3 = mr[:, :, :, :N].reshape(B, C, N)

    Wg = W[:, :, 0, 0].reshape(_GROUPS, COUT // _GROUPS, (2 * C) // _GROUPS)
    Wx = _block_diag(Wg[:, :, 0::2])
    Wj = _block_diag(Wg[:, :, 1::2])
    order = np.empty(C, dtype=np.int32)
    for h in range(_NS):
        for w in range(_WPT):
            order[h * 8 + w] = h * 8 + 2 * w
            order[h * 8 + _WPT + w] = h * 8 + 2 * w + 1
    Wj = Wj[:, order]

    out = _tc_conv(x3, mr3, Wx, Wj, bias)
    return out[..., None]


# speedup vs baseline: 2.0363x; 1.0452x over previous
import functools

import numpy as np

import jax
import jax.numpy as jnp
from jax import lax
from jax.experimental import pallas as pl
from jax.experimental.pallas import tpu as pltpu
from jax.experimental.pallas import tpu_sc as plsc

_GROUPS = 4
_LANES = 16
_NC, _NS = 2, 16
_WPT = 4


def _sc_maxrel(xTt, ijT, iiT, B, N, C, K, NP):
    S = 2000
    NCH = NP // S

    mesh = plsc.VectorSubcoreMesh(core_axis_name="c", subcore_axis_name="s")

    @functools.partial(
        pl.kernel,
        mesh=mesh,
        compiler_params=pltpu.CompilerParams(
            use_tc_tiling_on_sc=False, needs_layout_passes=False),
        out_type=jax.ShapeDtypeStruct((B, _NS, 2 * _WPT, NP), jnp.float32),
        scratch_types=[
            pltpu.VMEM((N * _WPT,), jnp.int32),
            pltpu.VMEM((K, S), jnp.int32),
            pltpu.VMEM((K, S), jnp.int32),
            pltpu.VMEM((2 * _WPT, S), jnp.float32),
            pltpu.SemaphoreType.DMA,
        ],
    )
    def sc_kernel(xTt_hbm, ij_hbm, ii_hbm, out_hbm, tbl_v, ij_v, ii_v, o_v, sem):
        b = lax.axis_index("c")
        h = lax.axis_index("s")

        pltpu.sync_copy(xTt_hbm.at[b, h], tbl_v)

        MASK = jnp.int32(-65536)

        def halves(v):
            lo = lax.bitcast_convert_type(v << 16, jnp.float32)
            hi = lax.bitcast_convert_type(v & MASK, jnp.float32)
            return lo, hi

        def chunk(t, carry):
            n0 = pl.multiple_of(t * S, S)
            pltpu.sync_copy(ij_hbm.at[b, :, pl.ds(n0, S)], ij_v)
            pltpu.sync_copy(ii_hbm.at[b, :, pl.ds(n0, S)], ii_v)

            def grp(g, c2):
                sl = pl.ds(pl.multiple_of(g * _LANES, _LANES), _LANES)
                acc_e = [None] * _WPT
                acc_o = [None] * _WPT
                for k in range(K):
                    bj = ij_v[k, sl] * _WPT
                    bi = ii_v[k, sl] * _WPT
                    for w in range(_WPT):
                        vj = plsc.load_gather(tbl_v, [bj + w])
                        vi = plsc.load_gather(tbl_v, [bi + w])
                        je, jo = halves(vj)
                        ie, io = halves(vi)
                        de = je - ie
                        do = jo - io
                        if k == 0:
                            acc_e[w] = de
                            acc_o[w] = do
                        else:
                            acc_e[w] = jnp.maximum(acc_e[w], de)
                            acc_o[w] = jnp.maximum(acc_o[w], do)
                for w in range(_WPT):
                    o_v[w, sl] = acc_e[w]
                    o_v[_WPT + w, sl] = acc_o[w]
                return c2

            lax.fori_loop(0, S // _LANES, grp, 0)
            pltpu.sync_copy(o_v, out_hbm.at[b, h, :, pl.ds(n0, S)])
            return carry

        lax.fori_loop(0, NCH, chunk, 0)

    return sc_kernel(xTt, ijT, iiT)


def _tc_body(wx_ref, wj_ref, b_ref, x_ref, mr_ref, o_ref):
    xb = x_ref[0]
    mr = mr_ref[0]
    acc = jnp.dot(wx_ref[...], xb, preferred_element_type=jnp.float32)
    acc = acc + jnp.dot(wj_ref[...], mr, preferred_element_type=jnp.float32)
    o_ref[0] = jnp.maximum(acc + b_ref[...], 0.0)


def _tc_conv(x3, mr3, Wx, Wj, bias):
    B, C, N = x3.shape
    COUT = Wx.shape[0]
    return pl.pallas_call(
        _tc_body,
        grid=(B,),
        in_specs=[
            pl.BlockSpec((COUT, C), lambda b: (0, 0)),
            pl.BlockSpec((COUT, C), lambda b: (0, 0)),
            pl.BlockSpec((COUT, 1), lambda b: (0, 0)),
            pl.BlockSpec((1, C, N), lambda b: (b, 0, 0)),
            pl.BlockSpec((1, C, N), lambda b: (b, 0, 0)),
        ],
        out_specs=pl.BlockSpec((1, COUT, N), lambda b: (b, 0, 0)),
        out_shape=jax.ShapeDtypeStruct((B, COUT, N), jnp.float32),
    )(Wx, Wj, bias.reshape(COUT, 1), x3, mr3)


def _block_diag(blocks):
    G, R, S = blocks.shape
    out = jnp.zeros((G * R, G * S), blocks.dtype)
    for g in range(G):
        out = out.at[g * R:(g + 1) * R, g * S:(g + 1) * S].set(blocks[g])
    return out


def kernel(x, edge_index, W, bias):
    B, C, N, _ = x.shape
    K = edge_index.shape[-1]
    COUT = W.shape[0]
    S = 2000
    NP = -(-N // S) * S

    x3 = x[..., 0]
    xT = jnp.transpose(x3, (0, 2, 1))
    xT32 = lax.bitcast_convert_type(
        xT.astype(jnp.bfloat16).reshape(B, N, C // 2, 2), jnp.int32)
    xTt = (xT32.reshape(B, N, _NS, _WPT)
           .transpose(0, 2, 1, 3).reshape(B, _NS, N * _WPT))

    ei = edge_index.astype(jnp.int32)
    pad = [(0, 0), (0, NP - N), (0, 0)]
    ijT = jnp.transpose(jnp.pad(ei[0], pad), (0, 2, 1))
    iiT = jnp.transpose(jnp.pad(ei[1], pad), (0, 2, 1))

    mr = _sc_maxrel(xTt, ijT, iiT, B, N, C, K, NP)
    mr3 = mr[:, :, :, :N].reshape(B, C, N)

    Wg = W[:, :, 0, 0].reshape(_GROUPS, COUT // _GROUPS, (2 * C) // _GROUPS)
    Wx = _block_diag(Wg[:, :, 0::2])
    Wj = _block_diag(Wg[:, :, 1::2])
    order = np.empty(C, dtype=np.int32)
    for h in range(_NS):
        for w in range(_WPT):
            order[h * 8 + w] = h * 8 + 2 * w
            order[h * 8 + _WPT + w] = h * 8 + 2 * w + 1
    Wj = Wj[:, order]

    out = _tc_conv(x3, mr3, Wx, Wj, bias)
    return out[..., None]


# speedup vs baseline: 2.0372x; 1.0005x over previous
import functools

import numpy as np

import jax
import jax.numpy as jnp
from jax import lax
from jax.experimental import pallas as pl
from jax.experimental.pallas import tpu as pltpu
from jax.experimental.pallas import tpu_sc as plsc

_GROUPS = 4
_LANES = 16
_NC, _NS = 2, 16
_WPT = 4


def _sc_maxrel(xTt, ijT, iiT, B, N, C, K, NP):
    S = 2000
    NCH = NP // S

    mesh = plsc.VectorSubcoreMesh(core_axis_name="c", subcore_axis_name="s")

    @functools.partial(
        pl.kernel,
        mesh=mesh,
        compiler_params=pltpu.CompilerParams(
            use_tc_tiling_on_sc=False, needs_layout_passes=False),
        out_type=jax.ShapeDtypeStruct((B, _NS, 2 * _WPT, NP), jnp.float32),
        scratch_types=[
            pltpu.VMEM((N * _WPT,), jnp.int32),
            pltpu.VMEM((K, S), jnp.int32),
            pltpu.VMEM((K, S), jnp.int32),
            pltpu.VMEM((2 * _WPT, S), jnp.float32),
            pltpu.SemaphoreType.DMA,
        ],
    )
    def sc_kernel(xTt_hbm, ij_hbm, ii_hbm, out_hbm, tbl_v, ij_v, ii_v, o_v, sem):
        b = lax.axis_index("c")
        h = lax.axis_index("s")

        pltpu.sync_copy(xTt_hbm.at[b, h], tbl_v)

        MASK = jnp.int32(-65536)

        def halves(v):
            lo = lax.bitcast_convert_type(v << 16, jnp.float32)
            hi = lax.bitcast_convert_type(v & MASK, jnp.float32)
            return lo, hi

        def chunk(t, carry):
            n0 = pl.multiple_of(t * S, S)
            pltpu.sync_copy(ij_hbm.at[b, :, pl.ds(n0, S)], ij_v)
            pltpu.sync_copy(ii_hbm.at[b, :, pl.ds(n0, S)], ii_v)

            def grp(g, c2):
                sl = pl.ds(pl.multiple_of(g * _LANES, _LANES), _LANES)
                acc_e = [None] * _WPT
                acc_o = [None] * _WPT
                for k in range(K):
                    bj = ij_v[k, sl] * _WPT
                    bi = ii_v[k, sl] * _WPT
                    for w in range(_WPT):
                        vj = plsc.load_gather(tbl_v, [bj + w])
                        vi = plsc.load_gather(tbl_v, [bi + w])
                        je, jo = halves(vj)
                        ie, io = halves(vi)
                        de = je - ie
                        do = jo - io
                        if k == 0:
                            acc_e[w] = de
                            acc_o[w] = do
                        else:
                            acc_e[w] = jnp.maximum(acc_e[w], de)
                            acc_o[w] = jnp.maximum(acc_o[w], do)
                for w in range(_WPT):
                    o_v[w, sl] = acc_e[w]
                    o_v[_WPT + w, sl] = acc_o[w]
                return c2

            lax.fori_loop(0, S // _LANES, grp, 0)
            pltpu.sync_copy(o_v, out_hbm.at[b, h, :, pl.ds(n0, S)])
            return carry

        lax.fori_loop(0, NCH, chunk, 0)

    return sc_kernel(xTt, ijT, iiT)


def _tc_xpart_body(wx_ref, b_ref, x_ref, o_ref):
    acc = jnp.dot(wx_ref[...], x_ref[0], preferred_element_type=jnp.float32)
    o_ref[0] = acc + b_ref[...]


def _tc_final_body(wj_ref, p_ref, mr_ref, o_ref):
    acc = jnp.dot(wj_ref[...], mr_ref[0], preferred_element_type=jnp.float32)
    o_ref[0] = jnp.maximum(acc + p_ref[0], 0.0)


def _tc_conv(x3, mr3, Wx, Wj, bias):
    B, C, N = x3.shape
    COUT = Wx.shape[0]
    part = pl.pallas_call(
        _tc_xpart_body,
        grid=(B,),
        in_specs=[
            pl.BlockSpec((COUT, C), lambda b: (0, 0)),
            pl.BlockSpec((COUT, 1), lambda b: (0, 0)),
            pl.BlockSpec((1, C, N), lambda b: (b, 0, 0)),
        ],
        out_specs=pl.BlockSpec((1, COUT, N), lambda b: (b, 0, 0)),
        out_shape=jax.ShapeDtypeStruct((B, COUT, N), jnp.float32),
    )(Wx, bias.reshape(COUT, 1), x3)
    return pl.pallas_call(
        _tc_final_body,
        grid=(B,),
        in_specs=[
            pl.BlockSpec((COUT, C), lambda b: (0, 0)),
            pl.BlockSpec((1, COUT, N), lambda b: (b, 0, 0)),
            pl.BlockSpec((1, C, N), lambda b: (b, 0, 0)),
        ],
        out_specs=pl.BlockSpec((1, COUT, N), lambda b: (b, 0, 0)),
        out_shape=jax.ShapeDtypeStruct((B, COUT, N), jnp.float32),
    )(Wj, part, mr3)


def _block_diag(blocks):
    G, R, S = blocks.shape
    out = jnp.zeros((G * R, G * S), blocks.dtype)
    for g in range(G):
        out = out.at[g * R:(g + 1) * R, g * S:(g + 1) * S].set(blocks[g])
    return out


def kernel(x, edge_index, W, bias):
    B, C, N, _ = x.shape
    K = edge_index.shape[-1]
    COUT = W.shape[0]
    S = 2000
    NP = -(-N // S) * S

    x3 = x[..., 0]
    xT = jnp.transpose(x3, (0, 2, 1))
    xT32 = lax.bitcast_convert_type(
        xT.astype(jnp.bfloat16).reshape(B, N, C // 2, 2), jnp.int32)
    xTt = (xT32.reshape(B, N, _NS, _WPT)
           .transpose(0, 2, 1, 3).reshape(B, _NS, N * _WPT))

    ei = edge_index.astype(jnp.int32)
    pad = [(0, 0), (0, NP - N), (0, 0)]
    ijT = jnp.transpose(jnp.pad(ei[0], pad), (0, 2, 1))
    iiT = jnp.transpose(jnp.pad(ei[1], pad), (0, 2, 1))

    mr = _sc_maxrel(xTt, ijT, iiT, B, N, C, K, NP)
    mr3 = mr[:, :, :, :N].reshape(B, C, N)

    Wg = W[:, :, 0, 0].reshape(_GROUPS, COUT // _GROUPS, (2 * C) // _GROUPS)
    Wx = _block_diag(Wg[:, :, 0::2])
    Wj = _block_diag(Wg[:, :, 1::2])
    order = np.empty(C, dtype=np.int32)
    for h in range(_NS):
        for w in range(_WPT):
            order[h * 8 + w] = h * 8 + 2 * w
            order[h * 8 + _WPT + w] = h * 8 + 2 * w + 1
    Wj = Wj[:, order]

    out = _tc_conv(x3, mr3, Wx, Wj, bias)
    return out[..., None]


# speedup vs baseline: 2.0382x; 1.0005x over previous
import functools

import numpy as np

import jax
import jax.numpy as jnp
from jax import lax
from jax.experimental import pallas as pl
from jax.experimental.pallas import tpu as pltpu
from jax.experimental.pallas import tpu_sc as plsc

_GROUPS = 4
_LANES = 16
_NC, _NS = 2, 16
_WPT = 4


def _sc_maxrel(xTt, ijT, iiT, B, N, C, K, NP):
    S = 2000
    NCH = NP // S

    mesh = plsc.VectorSubcoreMesh(core_axis_name="c", subcore_axis_name="s")

    @functools.partial(
        pl.kernel,
        mesh=mesh,
        compiler_params=pltpu.CompilerParams(
            use_tc_tiling_on_sc=False, needs_layout_passes=False),
        out_type=jax.ShapeDtypeStruct((B, _NS, 2 * _WPT, NP), jnp.float32),
        scratch_types=[
            pltpu.VMEM((N * _WPT,), jnp.int32),
            pltpu.VMEM((K, S), jnp.int32),
            pltpu.VMEM((K, S), jnp.int32),
            pltpu.VMEM((2 * _WPT, S), jnp.float32),
            pltpu.SemaphoreType.DMA,
        ],
    )
    def sc_kernel(xTt_hbm, ij_hbm, ii_hbm, out_hbm, tbl_v, ij_v, ii_v, o_v, sem):
        b = lax.axis_index("c")
        h = lax.axis_index("s")

        pltpu.sync_copy(xTt_hbm.at[b, h], tbl_v)

        MASK = jnp.int32(-65536)

        def halves(v):
            lo = lax.bitcast_convert_type(v << 16, jnp.float32)
            hi = lax.bitcast_convert_type(v & MASK, jnp.float32)
            return lo, hi

        def chunk(t, carry):
            n0 = pl.multiple_of(t * S, S)
            pltpu.sync_copy(ij_hbm.at[b, :, pl.ds(n0, S)], ij_v)
            pltpu.sync_copy(ii_hbm.at[b, :, pl.ds(n0, S)], ii_v)

            def grp(g, c2):
                sl = pl.ds(pl.multiple_of(g * _LANES, _LANES), _LANES)
                acc_e = [None] * _WPT
                acc_o = [None] * _WPT
                for k in range(K):
                    bj = ij_v[k, sl] * _WPT
                    bi = ii_v[k, sl] * _WPT
                    for w in range(_WPT):
                        vj = plsc.load_gather(tbl_v, [bj + w])
                        vi = plsc.load_gather(tbl_v, [bi + w])
                        je, jo = halves(vj)
                        ie, io = halves(vi)
                        de = je - ie
                        do = jo - io
                        if k == 0:
                            acc_e[w] = de
                            acc_o[w] = do
                        else:
                            acc_e[w] = jnp.maximum(acc_e[w], de)
                            acc_o[w] = jnp.maximum(acc_o[w], do)
                for w in range(_WPT):
                    o_v[w, sl] = acc_e[w]
                    o_v[_WPT + w, sl] = acc_o[w]
                return c2

            lax.fori_loop(0, S // _LANES, grp, 0)
            pltpu.sync_copy(o_v, out_hbm.at[b, h, :, pl.ds(n0, S)])
            return carry

        lax.fori_loop(0, NCH, chunk, 0)

    return sc_kernel(xTt, ijT, iiT)


def _tc_body(wx_ref, wj_ref, b_ref, x_ref, mr_ref, o_ref):
    xb = x_ref[0]
    mr = mr_ref[0]
    acc = jnp.dot(wx_ref[...], xb, preferred_element_type=jnp.float32)
    acc = acc + jnp.dot(wj_ref[...], mr, preferred_element_type=jnp.float32)
    o_ref[0] = jnp.maximum(acc + b_ref[...], 0.0)


def _tc_conv(x3, mr3, Wx, Wj, bias):
    B, C, N = x3.shape
    COUT = Wx.shape[0]
    return pl.pallas_call(
        _tc_body,
        grid=(B,),
        in_specs=[
            pl.BlockSpec((COUT, C), lambda b: (0, 0)),
            pl.BlockSpec((COUT, C), lambda b: (0, 0)),
            pl.BlockSpec((COUT, 1), lambda b: (0, 0)),
            pl.BlockSpec((1, C, N), lambda b: (b, 0, 0)),
            pl.BlockSpec((1, C, N), lambda b: (b, 0, 0)),
        ],
        out_specs=pl.BlockSpec((1, COUT, N), lambda b: (b, 0, 0)),
        out_shape=jax.ShapeDtypeStruct((B, COUT, N), jnp.float32),
    )(Wx, Wj, bias.reshape(COUT, 1), x3, mr3)


def _block_diag(blocks):
    G, R, S = blocks.shape
    out = jnp.zeros((G * R, G * S), blocks.dtype)
    for g in range(G):
        out = out.at[g * R:(g + 1) * R, g * S:(g + 1) * S].set(blocks[g])
    return out


def kernel(x, edge_index, W, bias):
    B, C, N, _ = x.shape
    K = edge_index.shape[-1]
    COUT = W.shape[0]
    S = 2000
    NP = -(-N // S) * S

    x3 = x[..., 0]
    xT = jnp.transpose(x3, (0, 2, 1))
    xT32 = lax.bitcast_convert_type(
        xT.astype(jnp.bfloat16).reshape(B, N, C // 2, 2), jnp.int32)
    xTt = (xT32.reshape(B, N, _NS, _WPT)
           .transpose(0, 2, 1, 3).reshape(B, _NS, N * _WPT))

    ei = edge_index.astype(jnp.int32)
    pad = [(0, 0), (0, NP - N), (0, 0)]
    ijT = jnp.transpose(jnp.pad(ei[0], pad), (0, 2, 1))
    iiT = jnp.transpose(jnp.pad(ei[1], pad), (0, 2, 1))

    mr = _sc_maxrel(xTt, ijT, iiT, B, N, C, K, NP)
    mr3 = mr[:, :, :, :N].reshape(B, C, N)

    Wg = W[:, :, 0, 0].reshape(_GROUPS, COUT // _GROUPS, (2 * C) // _GROUPS)
    Wx = _block_diag(Wg[:, :, 0::2])
    Wj = _block_diag(Wg[:, :, 1::2])
    order = np.empty(C, dtype=np.int32)
    for h in range(_NS):
        for w in range(_WPT):
            order[h * 8 + w] = h * 8 + 2 * w
            order[h * 8 + _WPT + w] = h * 8 + 2 * w + 1
    Wj = Wj[:, order]

    out = _tc_conv(x3, mr3, Wx, Wj, bias)
    return out[..., None]
